# baseline (speedup 1.0000x reference)
import functools

import jax
import jax.numpy as jnp
from jax.experimental import pallas as pl
from jax.experimental.pallas import tpu as pltpu

_TILE = 1024
_LANES = 128
_LOG2E = 1.4426950408889634
_LN2 = 0.6931471805599453

_DN_T = (((1,), (1,)), ((), ()))


def _antideriv_tanh(z):
    az = jnp.abs(z)
    t = jnp.exp2(az * (-2.0 * _LOG2E))
    return az + _LN2 * jnp.log2(1.0 + t)


def _act_terms(z):
    az = jnp.abs(z)
    zb = az.astype(jnp.bfloat16)
    t = jnp.exp2(zb * jnp.bfloat16(-2.0 * _LOG2E))
    sp = jnp.log2(1.0 + t.astype(jnp.float32))
    return az, sp


def _phi_body(r, n_blocks, m, x_ref, w0x_ref, wh_ref, b0_ref, bh_ref,
              ww_ref, cb_ref, out_ref):

    xb = x_ref[...].astype(jnp.bfloat16)

    zx = jax.lax.dot_general(xb, w0x_ref[...], _DN_T,
                             preferred_element_type=jnp.float32)
    z = zx[:, :m]
    xa = zx[:, m:]
    lane = jax.lax.broadcasted_iota(jnp.int32, (1, _LANES), 1)
    mask_a = (lane < r).astype(jnp.float32)
    mask_cw = (lane == r).astype(jnp.float32)
    quad_cw = jnp.sum(xa * (0.5 * xa * mask_a + mask_cw), axis=1,
                      keepdims=True)

    u = _antideriv_tanh(z + b0_ref[...])
    h = 1.0 / n_blocks
    for i in range(n_blocks):
        zi = jax.lax.dot_general(u, wh_ref[i], _DN_T,
                                 preferred_element_type=jnp.float32)
        u = u + h * _antideriv_tanh(zi + bh_ref[i:i + 1, :])

    uw = jnp.sum(u * ww_ref[...], axis=1, keepdims=True)
    out_ref[...] = uw + quad_cw + cb_ref[0:1, 0:1]


def kernel(x, A, W0, b0, Wh, bh, ww, cw, cb):
    nex, dp1 = x.shape
    m = W0.shape[0]
    n_blocks = Wh.shape[0]
    r = A.shape[0]

    acw = (jnp.zeros((_LANES, dp1), jnp.float32)
           .at[:r, :].set(A).at[r, :].set(cw[0]))
    w0x = jnp.concatenate([W0, acw], axis=0).astype(jnp.bfloat16)
    whb = Wh.astype(jnp.bfloat16)
    cbrow = jnp.full((1, _LANES), cb[0], jnp.float32)

    grid = (nex // _TILE,)
    body = functools.partial(_phi_body, r, n_blocks, m)
    padded = pl.pallas_call(
        body,
        grid=grid,
        in_specs=[
            pl.BlockSpec((_TILE, dp1), lambda i: (i, 0)),
            pl.BlockSpec((m + _LANES, dp1), lambda i: (0, 0)),
            pl.BlockSpec((n_blocks, m, m), lambda i: (0, 0, 0)),
            pl.BlockSpec((1, m), lambda i: (0, 0)),
            pl.BlockSpec((n_blocks, m), lambda i: (0, 0)),
            pl.BlockSpec((1, m), lambda i: (0, 0)),
            pl.BlockSpec((1, _LANES), lambda i: (0, 0)),
        ],
        out_specs=pl.BlockSpec((_TILE, 1), lambda i: (i, 0)),
        out_shape=jax.ShapeDtypeStruct((nex, 1), jnp.float32),
        compiler_params=pltpu.CompilerParams(
            dimension_semantics=("parallel",),
            vmem_limit_bytes=100 * 1024 * 1024,
        ),
    )(x, w0x, whb, b0[None], bh, ww, cbrow)
    return padded


# speedup vs baseline: 1.0012x; 1.0012x over previous
import functools

import jax
import jax.numpy as jnp
from jax.experimental import pallas as pl
from jax.experimental.pallas import tpu as pltpu

_TILE = 512
_LANES = 128
_LOG2E = 1.4426950408889634
_LN2 = 0.6931471805599453

_DN_T = (((1,), (1,)), ((), ()))


def _antideriv_tanh(z):
    az = jnp.abs(z)
    t = jnp.exp2(az * (-2.0 * _LOG2E))
    return az + _LN2 * jnp.log2(1.0 + t)


def _phi_body(r, n_blocks, m, x_ref, w0x_ref, wh_ref, b0_ref, bh_ref,
              ww_ref, cb_ref, out_ref):

    xb = x_ref[...].astype(jnp.bfloat16)

    zx = jax.lax.dot_general(xb, w0x_ref[...], _DN_T,
                             preferred_element_type=jnp.float32)
    z = zx[:, :m]
    xa = zx[:, m:]
    lane = jax.lax.broadcasted_iota(jnp.int32, (1, _LANES), 1)
    mask_a = (lane < r).astype(jnp.float32)
    mask_cw = (lane == r).astype(jnp.float32)
    quad_cw = jnp.sum(xa * (0.5 * xa * mask_a + mask_cw), axis=1,
                      keepdims=True)

    u = _antideriv_tanh(z + b0_ref[...])
    h = 1.0 / n_blocks
    for i in range(n_blocks):
        zi = jax.lax.dot_general(u, wh_ref[i], _DN_T,
                                 preferred_element_type=jnp.float32)
        u = u + h * _antideriv_tanh(zi + bh_ref[i:i + 1, :])

    uw = jnp.sum(u * ww_ref[...], axis=1, keepdims=True)
    out_ref[...] = uw + quad_cw + cb_ref[0:1, 0:1]


def kernel(x, A, W0, b0, Wh, bh, ww, cw, cb):
    nex, dp1 = x.shape
    m = W0.shape[0]
    n_blocks = Wh.shape[0]
    r = A.shape[0]

    acw = (jnp.zeros((_LANES, dp1), jnp.float32)
           .at[:r, :].set(A).at[r, :].set(cw[0]))
    w0x = jnp.concatenate([W0, acw], axis=0).astype(jnp.bfloat16)
    whb = Wh.astype(jnp.bfloat16)
    cbrow = jnp.full((1, _LANES), cb[0], jnp.float32)

    grid = (nex // _TILE,)
    body = functools.partial(_phi_body, r, n_blocks, m)
    padded = pl.pallas_call(
        body,
        grid=grid,
        in_specs=[
            pl.BlockSpec((_TILE, dp1), lambda i: (i, 0)),
            pl.BlockSpec((m + _LANES, dp1), lambda i: (0, 0)),
            pl.BlockSpec((n_blocks, m, m), lambda i: (0, 0, 0)),
            pl.BlockSpec((1, m), lambda i: (0, 0)),
            pl.BlockSpec((n_blocks, m), lambda i: (0, 0)),
            pl.BlockSpec((1, m), lambda i: (0, 0)),
            pl.BlockSpec((1, _LANES), lambda i: (0, 0)),
        ],
        out_specs=pl.BlockSpec((_TILE, 1), lambda i: (i, 0)),
        out_shape=jax.ShapeDtypeStruct((nex, 1), jnp.float32),
        compiler_params=pltpu.CompilerParams(
            dimension_semantics=("parallel",),
            vmem_limit_bytes=100 * 1024 * 1024,
        ),
    )(x, w0x, whb, b0[None], bh, ww, cbrow)
    return padded
